# baseline (speedup 1.0000x reference)
import functools

import jax
import jax.numpy as jnp
from jax.experimental import pallas as pl
from jax.experimental.pallas import tpu as pltpu

_NAT = 40
_NBOND = 4
_NBT = 7
_BN_F = 64
_K_F = 32
_N_BLOCKS = 1024
_CHUNK = 1000


def _elu(x):
    return jnp.where(x > 0, x, jnp.exp(jnp.minimum(x, 0.0)) - 1.0)


def _bnl_matmul(x, gamma, beta, W, b, apply_elu_pre=True):
    N, D = x.shape
    F = W.shape[1]
    grid = N // _CHUNK

    def body(x_ref, g_ref, be_ref, w_ref, b_ref, o_ref):
        a = x_ref[...]
        if apply_elu_pre:
            a = _elu(a * g_ref[...] + be_ref[...])
        o_ref[...] = (jnp.dot(a, w_ref[...], preferred_element_type=jnp.float32)
                      + b_ref[...])

    return pl.pallas_call(
        body,
        grid=(grid,),
        in_specs=[
            pl.BlockSpec((_CHUNK, D), lambda i: (i, 0)),
            pl.BlockSpec((1, D), lambda i: (0, 0)),
            pl.BlockSpec((1, D), lambda i: (0, 0)),
            pl.BlockSpec((D, F), lambda i: (0, 0)),
            pl.BlockSpec((1, F), lambda i: (0, 0)),
        ],
        out_specs=pl.BlockSpec((_CHUNK, F), lambda i: (i, 0)),
        out_shape=jax.ShapeDtypeStruct((N, F), jnp.float32),
    )(x, gamma.reshape(1, D), beta.reshape(1, D), W, b.reshape(1, F))


def _mlp3(z, W1, b1, W2, b2, W3, b3):
    N, D = z.shape
    H1 = W1.shape[1]
    H2 = W2.shape[1]
    F = W3.shape[1]
    grid = N // _CHUNK

    def body(z_ref, w1_ref, b1_ref, w2_ref, b2_ref, w3_ref, b3_ref, o_ref):
        t = _elu(jnp.dot(z_ref[...], w1_ref[...],
                         preferred_element_type=jnp.float32) + b1_ref[...])
        t = _elu(jnp.dot(t, w2_ref[...],
                         preferred_element_type=jnp.float32) + b2_ref[...])
        o_ref[...] = (jnp.dot(t, w3_ref[...],
                              preferred_element_type=jnp.float32) + b3_ref[...])

    const = lambda i: (0, 0)
    return pl.pallas_call(
        body,
        grid=(grid,),
        in_specs=[
            pl.BlockSpec((_CHUNK, D), lambda i: (i, 0)),
            pl.BlockSpec((D, H1), const), pl.BlockSpec((1, H1), const),
            pl.BlockSpec((H1, H2), const), pl.BlockSpec((1, H2), const),
            pl.BlockSpec((H2, F), const), pl.BlockSpec((1, F), const),
        ],
        out_specs=pl.BlockSpec((_CHUNK, F), lambda i: (i, 0)),
        out_shape=jax.ShapeDtypeStruct((N, F), jnp.float32),
    )(z, W1, b1.reshape(1, H1), W2, b2.reshape(1, H2), W3, b3.reshape(1, F))


def kernel(params, atom_types, is_scaffold, bond_info, block_ids, last_append_mask):
    n = atom_types.shape[0]
    begin = bond_info[:, 0]
    end = bond_info[:, 1]
    btype = bond_info[:, 2]

    at = jnp.where(is_scaffold == 1, atom_types + _NAT,
         jnp.where(last_append_mask == 1, atom_types + 2 * _NAT,
         jnp.where(last_append_mask == 2, atom_types + 3 * _NAT, atom_types)))
    at = jnp.where(is_scaffold == 1, at + _NAT, at)
    feats = jnp.take(params['emb'], at, axis=0)

    for lp in params['layers']:
        bn = lp['bn']
        h = _bnl_matmul(feats, bn['gamma'], bn['beta'], bn['W'], bn['b'])
        msgs = jnp.take(h, end, axis=0)
        agg = jnp.zeros((n, _NBT, _BN_F), jnp.float32).at[begin, btype].add(msgs)
        z = jnp.concatenate([h, agg.reshape(n, _NBT * _BN_F)], axis=-1)
        mlp = lp['mlp']
        z = _mlp3(z, mlp[0]['W'], mlp[0]['b'], mlp[1]['W'], mlp[1]['b'],
                  mlp[2]['W'], mlp[2]['b'])
        feats = jnp.concatenate([feats, z], axis=-1)

    fin = params['final']
    out = _bnl_matmul(feats, fin['gamma'], fin['beta'], fin['W'], fin['b'])
    hp = _elu(out * params['pool_gamma'] + params['pool_beta'])

    seg_sum = jax.ops.segment_sum(hp, block_ids, num_segments=_N_BLOCKS)
    cnt = jax.ops.segment_sum(jnp.ones((n,), jnp.float32), block_ids,
                              num_segments=_N_BLOCKS)
    mol = seg_sum / jnp.maximum(cnt, 1.0)[:, None]

    ac = params['append_connect']
    D1 = out.shape[1]
    U = _bnl_matmul(out, ac['gamma'][:D1], ac['beta'][:D1], ac['W'][:D1], ac['b'])
    Vsmall = _elu(mol * ac['gamma'][D1:] + ac['beta'][D1:]) @ ac['W'][D1:]
    act_ac = U + jnp.take(Vsmall, block_ids, axis=0)

    ep = params['end']
    act_end = (_elu(mol * ep['gamma'] + ep['beta']) @ ep['W'] + ep['b'])[:, 0]

    row_max = jnp.max(act_ac, axis=-1)
    seg_max = jax.ops.segment_max(row_max, block_ids, num_segments=_N_BLOCKS)
    m = jnp.maximum(seg_max, act_end)
    ex = jnp.exp(act_ac - jnp.take(m, block_ids)[:, None])
    eb = jnp.exp(act_end - m)
    Z = jax.ops.segment_sum(jnp.sum(ex, axis=-1), block_ids,
                            num_segments=_N_BLOCKS) + eb
    p_ac = ex / jnp.take(Z, block_ids)[:, None]
    p_end = eb / Z
    p_append = p_ac[:, :_NAT * _NBOND].reshape(n, _NAT, _NBOND)
    p_connect = p_ac[:, _NAT * _NBOND:]
    return (p_append, p_connect, p_end)
